# Initial kernel scaffold; baseline (speedup 1.0000x reference)
import jax
import jax.numpy as jnp
from jax.experimental import pallas as pl


def kernel(values, alpha, temp, gamma_hs, U, V, ln_w, ln_b):
    raise NotImplementedError("write your pallas kernel here")


# speedup vs baseline: 25.3305x; 25.3305x over previous
from math import sqrt

import jax
import jax.numpy as jnp
import numpy as np
from jax.experimental import pallas as pl

H, S, F = 16, 2048, 512
B, D = 2, 64
RANK = 12
K = max(1, int(0.1 * F))
S_BLK = 256
_MIN32 = -2147483648
_BITVALS = [(_MIN32 if b == 31 else (1 << b)) for b in range(31, -1, -1)]


def _score_kernel(values_ref, tempT_ref, ln_w_ref, ln_b_ref, score_ref):
    v = values_ref[...]
    energy = jnp.mean(v * v, axis=3)
    rms = jnp.maximum(jnp.sqrt(jnp.mean(energy, axis=1, keepdims=True)), 1e-6)
    score = energy / rms
    gain = jax.nn.softplus(tempT_ref[...])
    score = score * gain[None]
    mu = jnp.mean(score, axis=1, keepdims=True)
    var = jnp.mean((score - mu) ** 2, axis=1, keepdims=True)
    score = (score - mu) / jnp.sqrt(var + 1e-5)
    score_ref[...] = score * ln_w_ref[...][None] + ln_b_ref[...][None]


def _topk_softmax(L):
    bits = jax.lax.bitcast_convert_type(L, jnp.int32)
    keys = bits ^ ((bits >> 31) & jnp.int32(0x7FFFFFFF))
    rows = L.shape[0]
    prefix = jnp.zeros((rows, 1), jnp.int32)
    for bv in _BITVALS:
        cand = prefix | jnp.int32(bv)
        cand_s = cand ^ jnp.int32(_MIN32)
        cnt = jnp.sum((keys >= cand_s).astype(jnp.float32), axis=-1,
                      keepdims=True)
        prefix = jnp.where(cnt >= float(K), cand, prefix)
    thr = prefix ^ jnp.int32(_MIN32)
    mask = keys >= thr
    rowmax = jnp.max(L, axis=-1, keepdims=True)
    p = jnp.where(mask, jnp.exp(L - rowmax), 0.0)
    z = jnp.sum(p, axis=-1, keepdims=True)
    return p / z


def _attn_kernel(score_ref, alpha_ref, u_ref, v_ref, gamma_ref, w_ref,
                 out_ref):
    scale = 1.0 / sqrt(F)
    u = u_ref[0]
    vv = v_ref[0]
    bil = jnp.dot(u, vv, preferred_element_type=jnp.float32,
                  precision=jax.lax.Precision.HIGHEST)
    g = gamma_ref[0]
    base = bil + g
    l0 = base + score_ref[0, 0]
    l1 = base + score_ref[1, 0]
    la = alpha_ref[0] * scale
    L = jnp.concatenate([l0, l1, la], axis=0)
    attn = _topk_softmax(L)
    a_al = attn[2 * S_BLK:]
    a0 = attn[:S_BLK] + a_al
    a1 = attn[S_BLK:2 * S_BLK] + a_al
    w0 = w_ref[0, 0]
    w1 = w_ref[1, 0]
    out_ref[0, 0] = jnp.dot(a0, w0, preferred_element_type=jnp.float32,
                            precision=jax.lax.Precision.HIGHEST)
    out_ref[0, 1] = jnp.dot(a1, w1, preferred_element_type=jnp.float32,
                            precision=jax.lax.Precision.HIGHEST)


def kernel(values, alpha, temp, gamma_hs, U, V, ln_w, ln_b):
    score_bfh = pl.pallas_call(
        _score_kernel,
        out_shape=jax.ShapeDtypeStruct((B, F, H), jnp.float32),
    )(values, temp.reshape(1, H), ln_w.reshape(F, 1), ln_b.reshape(F, 1))
    score4 = jnp.transpose(score_bfh, (0, 2, 1)).reshape(B, H, 1, F)
    values_t = jnp.transpose(values, (0, 2, 1, 3))

    out_hbsd = pl.pallas_call(
        _attn_kernel,
        grid=(H, S // S_BLK),
        in_specs=[
            pl.BlockSpec((B, 1, 1, F), lambda h, i: (0, h, 0, 0)),
            pl.BlockSpec((1, S_BLK, F), lambda h, i: (h, i, 0)),
            pl.BlockSpec((1, S_BLK, RANK), lambda h, i: (h, i, 0)),
            pl.BlockSpec((1, RANK, F), lambda h, i: (h, 0, 0)),
            pl.BlockSpec((1, S_BLK, 1), lambda h, i: (h, i, 0)),
            pl.BlockSpec((B, 1, F, D), lambda h, i: (0, h, 0, 0)),
        ],
        out_specs=pl.BlockSpec((1, B, S_BLK, D), lambda h, i: (h, 0, i, 0)),
        out_shape=jax.ShapeDtypeStruct((H, B, S, D), jnp.float32),
    )(score4, alpha, U, V, gamma_hs, values_t)
    return jnp.transpose(out_hbsd, (1, 2, 0, 3))
